# baseline (speedup 1.0000x reference)
import jax
import jax.numpy as jnp
from jax.experimental import pallas as pl

_B, _N = 4096, 8192
_K_OUT = 101
_D0 = 64
_LANES = 128


def _batcher_pairs(n):
    pairs = []
    p = 1
    while p < n:
        k = p
        while k >= 1:
            for j in range(k % p, n - k, 2 * k):
                for i in range(0, min(k, n - j - k)):
                    if (i + j) // (p * 2) == (i + j + k) // (p * 2):
                        pairs.append((i + j, i + j + k))
            k //= 2
        p *= 2
    return pairs


_LEAF_PAIRS = _batcher_pairs(_D0)


def _split_half(a):
    if a.shape[0] > 1:
        h = a.shape[0] // 2
        return a[:h], a[h:]
    s = a.shape[1] // 2
    return a[:, :s], a[:, s:]


def _rounds(zs, split):
    n = len(zs)
    if split:
        zs = [list(_split_half(a)) for a in zs]
    else:
        zs = [[a] for a in zs]
    d = n // 2
    while d >= 1:
        for i in range(n):
            if (i % (2 * d)) < d:
                j = i + d
                for pidx in range(len(zs[i])):
                    a, b = zs[i][pidx], zs[j][pidx]
                    hi = jnp.maximum(a, b)
                    lo = jnp.minimum(a, b)
                    if split and pidx == 1:
                        zs[i][pidx], zs[j][pidx] = lo, hi
                    else:
                        zs[i][pidx], zs[j][pidx] = hi, lo
        d //= 2
    return zs


def _block_kernel(lgT_ref, lbT_ref, out_lgT_ref, out_lbT_ref):
    lgT = lgT_ref[...]
    lbT = lbT_ref[...]

    pos_row = jnp.sum(lgT * lbT, axis=0, keepdims=True)
    lab_row = jnp.sum(lbT, axis=0, keepdims=True)

    ml = jnp.where(lbT != 0, jnp.float32(-jnp.inf), lgT)

    ps, qs = [], []
    for d in range(_D0):
        e = ml[d * 128 : (d + 1) * 128, :].reshape(16, 8, _LANES)
        ps.append(e[:8])
        qs.append(e[8:])
    for i, j in _LEAF_PAIRS:
        ps[i], ps[j] = jnp.maximum(ps[i], ps[j]), jnp.minimum(ps[i], ps[j])
        qs[i], qs[j] = jnp.minimum(qs[i], qs[j]), jnp.maximum(qs[i], qs[j])

    zs = _rounds(ps + qs, split=True)

    while True:
        merged = [jnp.maximum(e[0], e[1]) for e in zs]
        last = merged[0].shape == (1, 1, _LANES)
        zs = _rounds(merged, split=not last)
        if last:
            break

    top_rows = [zs[j][0].reshape(1, _LANES) for j in range(127)]
    out_lgT_ref[...] = jnp.concatenate([pos_row] + top_rows, axis=0)
    zero_rows = jnp.zeros((127, _LANES), jnp.float32)
    out_lbT_ref[...] = jnp.concatenate([lab_row, zero_rows], axis=0)


_CHUNKS = 4


@jax.jit
def kernel(logits, labels):
    rows = _B // _CHUNKS
    in_spec = pl.BlockSpec((_N, _LANES), lambda i: (0, i))
    out_spec = pl.BlockSpec((128, _LANES), lambda i: (0, i))
    call = pl.pallas_call(
        _block_kernel,
        grid=(rows // _LANES,),
        in_specs=[in_spec, in_spec],
        out_specs=[out_spec, out_spec],
        out_shape=[
            jax.ShapeDtypeStruct((128, rows), jnp.float32),
            jax.ShapeDtypeStruct((128, rows), jnp.float32),
        ],
    )
    lg_parts, lb_parts = [], []
    for c in range(_CHUNKS):
        lgT = logits[c * rows : (c + 1) * rows].T
        lbT = labels[c * rows : (c + 1) * rows].T
        out_lgT, out_lbT = call(lgT, lbT)
        lg_parts.append(out_lgT)
        lb_parts.append(out_lbT)
    out_lgT = jnp.concatenate(lg_parts, axis=1)
    out_lbT = jnp.concatenate(lb_parts, axis=1)
    return (out_lgT.T[:, :_K_OUT], out_lbT.T[:, :_K_OUT])


# speedup vs baseline: 1.7819x; 1.7819x over previous
import jax
import jax.numpy as jnp
from jax.experimental import pallas as pl

_B, _N = 4096, 8192
_K_OUT = 101
_D0 = 64
_LANES = 128


def _batcher_pairs(n):
    pairs = []
    p = 1
    while p < n:
        k = p
        while k >= 1:
            for j in range(k % p, n - k, 2 * k):
                for i in range(0, min(k, n - j - k)):
                    if (i + j) // (p * 2) == (i + j + k) // (p * 2):
                        pairs.append((i + j, i + j + k))
            k //= 2
        p *= 2
    return pairs


_LEAF_PAIRS = _batcher_pairs(_D0)


def _split_half(a):
    if a.shape[0] > 1:
        h = a.shape[0] // 2
        return a[:h], a[h:]
    s = a.shape[1] // 2
    return a[:, :s], a[:, s:]


def _rounds(zs, split):
    n = len(zs)
    if split:
        zs = [list(_split_half(a)) for a in zs]
    else:
        zs = [[a] for a in zs]
    d = n // 2
    while d >= 1:
        for i in range(n):
            if (i % (2 * d)) < d:
                j = i + d
                for pidx in range(len(zs[i])):
                    a, b = zs[i][pidx], zs[j][pidx]
                    hi = jnp.maximum(a, b)
                    lo = jnp.minimum(a, b)
                    if split and pidx == 1:
                        zs[i][pidx], zs[j][pidx] = lo, hi
                    else:
                        zs[i][pidx], zs[j][pidx] = hi, lo
        d //= 2
    return zs


def _block_kernel(lgT_ref, lg_ref, lb_ref, out_lgT_ref, out_lbT_ref):
    lgT = lgT_ref[...]
    lg = lg_ref[...]
    lb = lb_ref[...]

    pos_col = jnp.sum(lg * lb, axis=1, keepdims=True)
    lab_col = jnp.sum(lb, axis=1, keepdims=True)
    pos_row = jnp.swapaxes(pos_col, 0, 1)
    lab_row = jnp.swapaxes(lab_col, 0, 1)

    ps, qs = [], []
    for d in range(_D0):
        e = lgT[d * 128 : (d + 1) * 128, :].reshape(16, 8, _LANES)
        ps.append(e[:8])
        qs.append(e[8:])
    for i, j in _LEAF_PAIRS:
        ps[i], ps[j] = jnp.maximum(ps[i], ps[j]), jnp.minimum(ps[i], ps[j])
        qs[i], qs[j] = jnp.minimum(qs[i], qs[j]), jnp.maximum(qs[i], qs[j])

    zs = _rounds(ps + qs, split=True)

    while True:
        merged = [jnp.maximum(e[0], e[1]) for e in zs]
        last = merged[0].shape == (1, 1, _LANES)
        zs = _rounds(merged, split=not last)
        if last:
            break

    s = [zs[j][0].reshape(1, _LANES) for j in range(128)]
    g = s[127] <= pos_row
    top_rows = [
        jnp.where(g & (s[j - 1] <= pos_row), s[j], s[j - 1])
        for j in range(1, 128)
    ]
    out_lgT_ref[...] = jnp.concatenate([pos_row] + top_rows, axis=0)
    zero_rows = jnp.zeros((127, _LANES), jnp.float32)
    out_lbT_ref[...] = jnp.concatenate([lab_row, zero_rows], axis=0)


@jax.jit
def kernel(logits, labels):
    lgT = logits.T
    in_specT = pl.BlockSpec((_N, _LANES), lambda i: (0, i))
    in_spec_nat = pl.BlockSpec((_LANES, _N), lambda i: (i, 0))
    out_spec = pl.BlockSpec((128, _LANES), lambda i: (0, i))
    out_lgT, out_lbT = pl.pallas_call(
        _block_kernel,
        grid=(_B // _LANES,),
        in_specs=[in_specT, in_spec_nat, in_spec_nat],
        out_specs=[out_spec, out_spec],
        out_shape=[
            jax.ShapeDtypeStruct((128, _B), jnp.float32),
            jax.ShapeDtypeStruct((128, _B), jnp.float32),
        ],
    )(lgT, logits, labels)
    return (out_lgT.T[:, :_K_OUT], out_lbT.T[:, :_K_OUT])
